# baseline (speedup 1.0000x reference)
import math

import jax
import jax.numpy as jnp
from jax.experimental import pallas as pl
from jax.experimental.pallas import tpu as pltpu

_V = 8192
_S = 2048
_B = 4
_PAD = 0
_SMOOTH = 0.1
_CONF = 1.0 - _SMOOTH
_EMO_W = 5.0
_EPS = _SMOOTH / (_V - 1)
_CENT = (_V - 1) * _EPS * math.log(_EPS) + _CONF * math.log(_CONF)
_LAM = (_CONF - _EPS) / _EPS
_R = 256


def _loss_kernel(emo_ref, t_ref, x_ref, loss_ref, emo_loss_ref, acc_ref):
    r = pl.program_id(0)
    nr = pl.num_programs(0)

    @pl.when(r == 0)
    def _init():
        acc_ref[0] = 0.0
        acc_ref[1] = 0.0
        acc_ref[2] = 0.0

    t_blk = t_ref[0]
    lanes = jax.lax.broadcasted_iota(jnp.int32, (_R, 128), 1)
    c = x_ref[:, pl.ds(0, 128)]
    m_acc = c
    sw_acc = c * jnp.where(lanes == t_blk, 1.0 + _LAM, 1.0)
    for k in range(1, _V // 128):
        c = x_ref[:, pl.ds(k * 128, 128)]
        m_acc = jnp.maximum(m_acc, c)
        w = jnp.where(lanes + (k * 128) == t_blk, 1.0 + _LAM, 1.0)
        sw_acc = sw_acc + c * w
    rmax = jnp.max(m_acc, axis=1, keepdims=True)
    sumw = jnp.sum(sw_acc, axis=1, keepdims=True)
    sumexp = jnp.sum(jnp.exp(x_ref[...] - rmax), axis=1, keepdims=True)
    lse = rmax + jnp.log(sumexp)
    vals = _CENT + lse - _EPS * sumw

    ignore = t_blk == _PAD
    row0 = r * _R
    b = row0 // _S
    s_pos = row0 % _S + jax.lax.broadcasted_iota(jnp.int32, (_R, 1), 0)
    em = s_pos == emo_ref[b]
    ew = jnp.where(em, _EMO_W, 1.0)
    acc_ref[0] += jnp.sum(jnp.where(ignore, 0.0, vals * ew))
    vm = jnp.where(ignore, 0.0, vals)
    ev = jnp.where(em, vm, 0.0)
    acc_ref[1] += jnp.sum(ev)
    acc_ref[2] += jnp.sum(jnp.where(em & (ev != 0.0), 1.0, 0.0))

    @pl.when(r == nr - 1)
    def _fin():
        loss_ref[0, 0] = acc_ref[0] / _B
        cnt = acc_ref[2]
        emo_loss_ref[0, 0] = jnp.where(
            cnt > 0.0, acc_ref[1] / jnp.maximum(cnt, 1.0), 0.0)


def kernel(x, target, emo_positions):
    B, S, V = x.shape
    N = B * S
    nr = N // _R
    x2 = x.reshape(N, V)
    t3 = target.reshape(nr, _R, 1).astype(jnp.int32)
    emo = emo_positions.astype(jnp.int32)

    loss, emo_loss = pl.pallas_call(
        _loss_kernel,
        grid=(nr,),
        in_specs=[
            pl.BlockSpec(memory_space=pltpu.SMEM),
            pl.BlockSpec((1, _R, 1), lambda r: (r, 0, 0)),
            pl.BlockSpec((_R, V), lambda r: (r, 0)),
        ],
        out_specs=[
            pl.BlockSpec(memory_space=pltpu.SMEM),
            pl.BlockSpec(memory_space=pltpu.SMEM),
        ],
        out_shape=[
            jax.ShapeDtypeStruct((1, 1), jnp.float32),
            jax.ShapeDtypeStruct((1, 1), jnp.float32),
        ],
        scratch_shapes=[pltpu.SMEM((3,), jnp.float32)],
        compiler_params=pltpu.CompilerParams(
            dimension_semantics=("arbitrary",),
        ),
    )(emo, t3, x2)
    return (loss[0, 0], emo_loss[0, 0])


# speedup vs baseline: 1.1035x; 1.1035x over previous
import math

import jax
import jax.numpy as jnp
from jax.experimental import pallas as pl
from jax.experimental.pallas import tpu as pltpu

_V = 8192
_S = 2048
_B = 4
_PAD = 0
_SMOOTH = 0.1
_CONF = 1.0 - _SMOOTH
_EMO_W = 5.0
_EPS = _SMOOTH / (_V - 1)
_CENT = (_V - 1) * _EPS * math.log(_EPS) + _CONF * math.log(_CONF)
_LAM = (_CONF - _EPS) / _EPS
_R = 512


def _loss_kernel(emo_ref, t_ref, x_ref, loss_ref, emo_loss_ref, acc_ref):
    r = pl.program_id(0)
    nr = pl.num_programs(0)

    @pl.when(r == 0)
    def _init():
        acc_ref[0] = 0.0
        acc_ref[1] = 0.0
        acc_ref[2] = 0.0

    xb = x_ref[...]
    t_blk = t_ref[0]
    rmax = jnp.max(xb, axis=1, keepdims=True)
    sumexp = jnp.sum(jnp.exp(xb - rmax), axis=1, keepdims=True)
    cols = jax.lax.broadcasted_iota(jnp.int32, (_R, _V), 1)
    w = jnp.where(cols == t_blk, 1.0 + _LAM, 1.0)
    sumw = jnp.sum(xb * w, axis=1, keepdims=True)
    lse = rmax + jnp.log(sumexp)
    vals = _CENT + lse - _EPS * sumw

    ignore = t_blk == _PAD
    row0 = r * _R
    b = row0 // _S
    s_pos = row0 % _S + jax.lax.broadcasted_iota(jnp.int32, (_R, 1), 0)
    em = s_pos == emo_ref[b]
    ew = jnp.where(em, _EMO_W, 1.0)
    acc_ref[0] += jnp.sum(jnp.where(ignore, 0.0, vals * ew))
    vm = jnp.where(ignore, 0.0, vals)
    ev = jnp.where(em, vm, 0.0)
    acc_ref[1] += jnp.sum(ev)
    acc_ref[2] += jnp.sum(jnp.where(em & (ev != 0.0), 1.0, 0.0))

    @pl.when(r == nr - 1)
    def _fin():
        loss_ref[0, 0] = acc_ref[0] / _B
        cnt = acc_ref[2]
        emo_loss_ref[0, 0] = jnp.where(
            cnt > 0.0, acc_ref[1] / jnp.maximum(cnt, 1.0), 0.0)


def kernel(x, target, emo_positions):
    B, S, V = x.shape
    N = B * S
    nr = N // _R
    x2 = x.reshape(N, V)
    t3 = target.reshape(nr, _R, 1).astype(jnp.int32)
    emo = emo_positions.astype(jnp.int32)

    loss, emo_loss = pl.pallas_call(
        _loss_kernel,
        grid=(nr,),
        in_specs=[
            pl.BlockSpec(memory_space=pltpu.SMEM),
            pl.BlockSpec((1, _R, 1), lambda r: (r, 0, 0)),
            pl.BlockSpec((_R, V), lambda r: (r, 0)),
        ],
        out_specs=[
            pl.BlockSpec(memory_space=pltpu.SMEM),
            pl.BlockSpec(memory_space=pltpu.SMEM),
        ],
        out_shape=[
            jax.ShapeDtypeStruct((1, 1), jnp.float32),
            jax.ShapeDtypeStruct((1, 1), jnp.float32),
        ],
        scratch_shapes=[pltpu.SMEM((3,), jnp.float32)],
        compiler_params=pltpu.CompilerParams(
            dimension_semantics=("arbitrary",),
        ),
    )(emo, t3, x2)
    return (loss[0, 0], emo_loss[0, 0])


# speedup vs baseline: 1.1884x; 1.0769x over previous
import math

import jax
import jax.numpy as jnp
from jax.experimental import pallas as pl
from jax.experimental.pallas import tpu as pltpu

_V = 8192
_S = 2048
_B = 4
_PAD = 0
_SMOOTH = 0.1
_CONF = 1.0 - _SMOOTH
_EMO_W = 5.0
_EPS = _SMOOTH / (_V - 1)
_CENT = (_V - 1) * _EPS * math.log(_EPS) + _CONF * math.log(_CONF)
_LAM = (_CONF - _EPS) / _EPS
_R = 512


def _loss_kernel(emo_ref, t_ref, x_ref, loss_ref, emo_loss_ref, acc_ref):
    r = pl.program_id(0)
    nr = pl.num_programs(0)

    @pl.when(r == 0)
    def _init():
        acc_ref[0] = 0.0
        acc_ref[1] = 0.0
        acc_ref[2] = 0.0

    xb = x_ref[...]
    t_blk = t_ref[0, 0, :].reshape(_R, 1)
    rmax = jnp.max(xb, axis=1, keepdims=True)
    sumexp = jnp.sum(jnp.exp(xb - rmax), axis=1, keepdims=True)
    cols = jax.lax.broadcasted_iota(jnp.int32, (_R, _V), 1)
    w = jnp.where(cols == t_blk, 1.0 + _LAM, 1.0)
    sumw = jnp.sum(xb * w, axis=1, keepdims=True)
    lse = rmax + jnp.log(sumexp)
    vals = _CENT + lse - _EPS * sumw

    ignore = t_blk == _PAD
    row0 = r * _R
    b = row0 // _S
    s_pos = row0 % _S + jax.lax.broadcasted_iota(jnp.int32, (_R, 1), 0)
    em = s_pos == emo_ref[b]
    ew = jnp.where(em, _EMO_W, 1.0)
    acc_ref[0] += jnp.sum(jnp.where(ignore, 0.0, vals * ew))
    vm = jnp.where(ignore, 0.0, vals)
    ev = jnp.where(em, vm, 0.0)
    acc_ref[1] += jnp.sum(ev)
    acc_ref[2] += jnp.sum(jnp.where(em & (ev != 0.0), 1.0, 0.0))

    @pl.when(r == nr - 1)
    def _fin():
        loss_ref[0, 0] = acc_ref[0] / _B
        cnt = acc_ref[2]
        emo_loss_ref[0, 0] = jnp.where(
            cnt > 0.0, acc_ref[1] / jnp.maximum(cnt, 1.0), 0.0)


def kernel(x, target, emo_positions):
    B, S, V = x.shape
    N = B * S
    nr = N // _R
    x2 = x.reshape(N, V)
    t3 = target.reshape(nr, 1, _R).astype(jnp.int32)
    emo = emo_positions.astype(jnp.int32)

    loss, emo_loss = pl.pallas_call(
        _loss_kernel,
        grid=(nr,),
        in_specs=[
            pl.BlockSpec(memory_space=pltpu.SMEM),
            pl.BlockSpec((1, 1, _R), lambda r: (r, 0, 0)),
            pl.BlockSpec((_R, V), lambda r: (r, 0)),
        ],
        out_specs=[
            pl.BlockSpec(memory_space=pltpu.SMEM),
            pl.BlockSpec(memory_space=pltpu.SMEM),
        ],
        out_shape=[
            jax.ShapeDtypeStruct((1, 1), jnp.float32),
            jax.ShapeDtypeStruct((1, 1), jnp.float32),
        ],
        scratch_shapes=[pltpu.SMEM((3,), jnp.float32)],
        compiler_params=pltpu.CompilerParams(
            dimension_semantics=("arbitrary",),
        ),
    )(emo, t3, x2)
    return (loss[0, 0], emo_loss[0, 0])
